# baseline (speedup 1.0000x reference)
import functools

import jax
import jax.numpy as jnp
from jax import lax
from jax.experimental import pallas as pl
from jax.experimental.pallas import tpu as pltpu
from jax.experimental.pallas import tpu_sc as plsc

_DIM = 128
_BB_DIM = 64
_B, _L = 4096, 200
_N = _B * _L
_NC, _NS = 2, 16
_NW = _NC * _NS
_CHUNK = 128
_NSTG = _N // 2
_PER_W_STG = _NSTG // _NW
_NCHS = _PER_W_STG // _CHUNK
_NBUF_SC = 4
_RPB = 16
_NB = _B // _RPB
_SENT = 120


def _sc_gather(idx_left, idx_right, table):
  mesh = plsc.VectorSubcoreMesh(
      core_axis_name="c", subcore_axis_name="s",
      num_cores=_NC, num_subcores=_NS)

  @functools.partial(
      pl.kernel,
      out_type=jax.ShapeDtypeStruct((_NSTG, _DIM), jnp.float32),
      mesh=mesh,
      compiler_params=pltpu.CompilerParams(use_tc_tiling_on_sc=False),
      scratch_types=(
          [pltpu.VMEM((_NCHS, _CHUNK), jnp.int32),
           pltpu.VMEM((_NCHS, _CHUNK), jnp.int32),
           pltpu.VMEM((_NBUF_SC, _CHUNK, _BB_DIM), jnp.float32),
           pltpu.VMEM((_NBUF_SC, _CHUNK, _BB_DIM), jnp.float32)]
          + [pltpu.SemaphoreType.DMA] * _NBUF_SC
      ),
  )
  def gather_kernel(idxl_hbm, idxr_hbm, table_hbm, out_hbm,
                    idxl_v, idxr_v, bufl, bufr, *gsems):
    wid = lax.axis_index("s") * _NC + lax.axis_index("c")
    pltpu.sync_copy(idxl_hbm.at[pl.ds(wid * _NCHS, _NCHS)], idxl_v)
    pltpu.sync_copy(idxr_hbm.at[pl.ds(wid * _NCHS, _NCHS)], idxr_v)

    def start_gather(c, b):
      pltpu.make_async_copy(
          table_hbm.at[idxl_v.at[c]], bufl.at[b], gsems[b]).start()
      pltpu.make_async_copy(
          table_hbm.at[idxr_v.at[c]], bufr.at[b], gsems[b]).start()

    def wait_gather(c, b):
      pltpu.make_async_copy(
          table_hbm.at[idxl_v.at[c]], bufl.at[b], gsems[b]).wait()
      pltpu.make_async_copy(
          table_hbm.at[idxr_v.at[c]], bufr.at[b], gsems[b]).wait()

    stg0 = wid * _PER_W_STG
    for b in range(_NBUF_SC):
      start_gather(b, b)

    def body(it, carry):
      c0 = it * _NBUF_SC
      for b in range(_NBUF_SC):
        c = c0 + b
        wait_gather(c, b)
        j0 = stg0 + c * _CHUNK
        pltpu.sync_copy(
            bufl.at[b], out_hbm.at[pl.ds(j0, _CHUNK), pl.ds(0, _BB_DIM)])
        pltpu.sync_copy(
            bufr.at[b],
            out_hbm.at[pl.ds(j0, _CHUNK), pl.ds(_BB_DIM, _BB_DIM)])

        @pl.when(c + _NBUF_SC < _NCHS)
        def _():
          start_gather(c + _NBUF_SC, b)
      return carry

    lax.fori_loop(0, _NCHS // _NBUF_SC, body, 0)

  return gather_kernel(idx_left, idx_right, table)


def _tc_body(cid_ref, stg_ref, comb_ref, wl_ref, wr_ref, pe_ref,
             iota_ref, rones_ref, h_ref, m_ref):
  cid_nat = cid_ref[...]
  m_ref[...] = jnp.where(cid_nat != 0, 0.0, -jnp.inf).astype(jnp.float32)

  cidT = jnp.transpose(cid_nat.astype(jnp.float32))
  bcast = jnp.dot(cidT, rones_ref[...],
                  preferred_element_type=jnp.float32)
  onehot = (bcast == iota_ref[...]).astype(jnp.float32)

  stg = stg_ref[...]
  projl = jnp.dot(stg, wl_ref[...], preferred_element_type=jnp.float32)
  projr = jnp.dot(stg, wr_ref[...], preferred_element_type=jnp.float32)

  pe = pe_ref[...]
  comb = comb_ref[...]
  for l in range(_RPB):
    base = jnp.dot(onehot[:, l * _DIM:(l + 1) * _DIM], comb,
                   preferred_element_type=jnp.float32)
    proj = (projl[(l % 8) * _L:(l % 8 + 1) * _L, :] if l < 8
            else projr[(l - 8) * _L:(l - 7) * _L, :])
    h_ref[l] = base + proj + pe


def _pe_table(L, d):
  pos = jnp.arange(L, dtype=jnp.float32)[:, None]
  i = jnp.arange(0, d, 2, dtype=jnp.float32)[None, :]
  angle = pos / jnp.power(10000.0, i / d)
  pe = jnp.zeros((L, d), dtype=jnp.float32)
  pe = pe.at[:, 0::2].set(jnp.sin(angle))
  pe = pe.at[:, 1::2].set(jnp.cos(angle))
  return pe


def kernel(token_types, bb_indices, rxn_indices, token_table, bb_table,
           bb_W, bb_b, rxn_table):
  tt32 = token_types.astype(jnp.int32)
  rx32 = rxn_indices.astype(jnp.int32)
  cidx = jnp.where(tt32 == 2, rx32 + 8,
                   jnp.where(tt32 == 1, jnp.int32(_SENT), tt32))

  masked_idx = jnp.where(tt32 == 1, bb_indices.astype(jnp.int32), 0)
  mi = masked_idx.reshape(_NB, 2, 8 * _L)
  idx_left = mi[:, 0].reshape(_NW * _NCHS, _CHUNK)
  idx_right = mi[:, 1].reshape(_NW * _NCHS, _CHUNK)

  staging = _sc_gather(idx_left, idx_right, bb_table)

  c0 = bb_table[0] @ bb_W
  comb = jnp.zeros((_DIM, _DIM), jnp.float32)
  comb = comb.at[:8].set(token_table)
  comb = comb.at[8:109].set(rxn_table)
  comb = comb.at[:109].add(-c0[None, :])
  comb = comb.at[_SENT].set(bb_b)

  wl = jnp.concatenate([bb_W, jnp.zeros((_BB_DIM, _DIM), jnp.float32)], 0)
  wr = jnp.concatenate([jnp.zeros((_BB_DIM, _DIM), jnp.float32), bb_W], 0)
  pe = _pe_table(_L, _DIM)
  iota_k = jnp.broadcast_to(
      jnp.tile(jnp.arange(_DIM, dtype=jnp.float32), _RPB)[None, :],
      (_L, _RPB * _DIM))
  rones = jnp.repeat(jnp.eye(_RPB, dtype=jnp.float32), _DIM, axis=1)

  h, m = pl.pallas_call(
      _tc_body,
      grid=(_NB,),
      in_specs=[
          pl.BlockSpec((_RPB, _L), lambda i: (i, 0)),
          pl.BlockSpec((8 * _L, _DIM), lambda i: (i, 0)),
          pl.BlockSpec((_DIM, _DIM), lambda i: (0, 0)),
          pl.BlockSpec((_DIM, _DIM), lambda i: (0, 0)),
          pl.BlockSpec((_DIM, _DIM), lambda i: (0, 0)),
          pl.BlockSpec((_L, _DIM), lambda i: (0, 0)),
          pl.BlockSpec((_L, _RPB * _DIM), lambda i: (0, 0)),
          pl.BlockSpec((_RPB, _RPB * _DIM), lambda i: (0, 0)),
      ],
      out_specs=[
          pl.BlockSpec((_RPB, _L, _DIM), lambda i: (i, 0, 0)),
          pl.BlockSpec((_RPB, _L), lambda i: (i, 0)),
      ],
      out_shape=[
          jax.ShapeDtypeStruct((_B, _L, _DIM), jnp.float32),
          jax.ShapeDtypeStruct((_B, _L), jnp.float32),
      ],
  )(cidx, staging, comb, wl, wr, pe, iota_k, rones)
  return h, m


# speedup vs baseline: 13.0803x; 13.0803x over previous
import functools

import jax
import jax.numpy as jnp
from jax import lax
from jax.experimental import pallas as pl
from jax.experimental.pallas import tpu as pltpu
from jax.experimental.pallas import tpu_sc as plsc

_DIM = 128
_BB_DIM = 64
_B, _L = 4096, 200
_N = _B * _L
_NC, _NS = 2, 16
_NW = _NC * _NS
_CHUNK = 128
_NSTG = _N // 2
_PER_W_STG = _NSTG // _NW
_NCHS = _PER_W_STG // _CHUNK
_NBUF_SC = 4
_RPB = 16
_NB = _B // _RPB
_SENT = 120


def _sc_gather(idx_left, idx_right, table):
  mesh = plsc.VectorSubcoreMesh(
      core_axis_name="c", subcore_axis_name="s",
      num_cores=_NC, num_subcores=_NS)

  @functools.partial(
      pl.kernel,
      out_type=jax.ShapeDtypeStruct((_NSTG, _DIM), jnp.float32),
      mesh=mesh,
      compiler_params=pltpu.CompilerParams(use_tc_tiling_on_sc=False),
      scratch_types=(
          [pltpu.VMEM((_NCHS, _CHUNK), jnp.int32),
           pltpu.VMEM((_NCHS, _CHUNK), jnp.int32),
           pltpu.VMEM((_NBUF_SC, _CHUNK, _BB_DIM), jnp.float32),
           pltpu.VMEM((_NBUF_SC, _CHUNK, _BB_DIM), jnp.float32)]
          + [pltpu.SemaphoreType.DMA] * _NBUF_SC
      ),
  )
  def gather_kernel(idxl_hbm, idxr_hbm, table_hbm, out_hbm,
                    idxl_v, idxr_v, bufl, bufr, *gsems):
    wid = lax.axis_index("s") * _NC + lax.axis_index("c")
    pltpu.sync_copy(idxl_hbm.at[pl.ds(wid * _NCHS, _NCHS)], idxl_v)
    pltpu.sync_copy(idxr_hbm.at[pl.ds(wid * _NCHS, _NCHS)], idxr_v)

    def start_gather(c, b):
      pltpu.make_async_copy(
          table_hbm.at[idxl_v.at[c]], bufl.at[b], gsems[b]).start()
      pltpu.make_async_copy(
          table_hbm.at[idxr_v.at[c]], bufr.at[b], gsems[b]).start()

    def wait_gather(c, b):
      pltpu.make_async_copy(
          table_hbm.at[idxl_v.at[c]], bufl.at[b], gsems[b]).wait()
      pltpu.make_async_copy(
          table_hbm.at[idxr_v.at[c]], bufr.at[b], gsems[b]).wait()

    stg0 = wid * _PER_W_STG
    for b in range(_NBUF_SC):
      start_gather(b, b)

    def body(it, carry):
      c0 = it * _NBUF_SC
      for b in range(_NBUF_SC):
        c = c0 + b
        wait_gather(c, b)
        j0 = stg0 + c * _CHUNK
        pltpu.sync_copy(
            bufl.at[b], out_hbm.at[pl.ds(j0, _CHUNK), pl.ds(0, _BB_DIM)])
        pltpu.sync_copy(
            bufr.at[b],
            out_hbm.at[pl.ds(j0, _CHUNK), pl.ds(_BB_DIM, _BB_DIM)])

        @pl.when(c + _NBUF_SC < _NCHS)
        def _():
          start_gather(c + _NBUF_SC, b)
      return carry

    lax.fori_loop(0, _NCHS // _NBUF_SC, body, 0)

  return gather_kernel(idx_left, idx_right, table)


def _tc_body(cid_ref, stg_ref, comb_ref, wl_ref, wr_ref, pe_ref,
             iota_ref, rones_ref, h_ref, m_ref):
  cid_nat = cid_ref[...]
  m_ref[...] = jnp.where(cid_nat != 0, 0.0, -jnp.inf).astype(jnp.float32)

  cidT = jnp.transpose(cid_nat.astype(jnp.float32))
  bcast = jnp.dot(cidT, rones_ref[...],
                  preferred_element_type=jnp.float32)
  onehot = (bcast == iota_ref[...]).astype(jnp.float32)

  stg = stg_ref[...]
  projl = jnp.dot(stg, wl_ref[...], preferred_element_type=jnp.float32)
  projr = jnp.dot(stg, wr_ref[...], preferred_element_type=jnp.float32)

  pe = pe_ref[...]
  comb = comb_ref[...]
  sent = jnp.float32(_SENT)
  for l in range(_RPB):
    base = jnp.dot(onehot[:, l * _DIM:(l + 1) * _DIM], comb,
                   preferred_element_type=jnp.float32)
    proj = (projl[(l % 8) * _L:(l % 8 + 1) * _L, :] if l < 8
            else projr[(l - 8) * _L:(l - 7) * _L, :])
    is_bb = bcast[:, l * _DIM:(l + 1) * _DIM] == sent
    h_ref[l] = base + pe + jnp.where(is_bb, proj, 0.0)


def _pe_table(L, d):
  pos = jnp.arange(L, dtype=jnp.float32)[:, None]
  i = jnp.arange(0, d, 2, dtype=jnp.float32)[None, :]
  angle = pos / jnp.power(10000.0, i / d)
  pe = jnp.zeros((L, d), dtype=jnp.float32)
  pe = pe.at[:, 0::2].set(jnp.sin(angle))
  pe = pe.at[:, 1::2].set(jnp.cos(angle))
  return pe


def kernel(token_types, bb_indices, rxn_indices, token_table, bb_table,
           bb_W, bb_b, rxn_table):
  tt32 = token_types.astype(jnp.int32)
  rx32 = rxn_indices.astype(jnp.int32)
  cidx = jnp.where(tt32 == 2, rx32 + 8,
                   jnp.where(tt32 == 1, jnp.int32(_SENT), tt32))

  mi = bb_indices.astype(jnp.int32).reshape(_NB, 2, 8 * _L)
  idx_left = mi[:, 0].reshape(_NW * _NCHS, _CHUNK)
  idx_right = mi[:, 1].reshape(_NW * _NCHS, _CHUNK)

  staging = _sc_gather(idx_left, idx_right, bb_table)

  comb = jnp.zeros((_DIM, _DIM), jnp.float32)
  comb = comb.at[:8].set(token_table)
  comb = comb.at[8:109].set(rxn_table)
  comb = comb.at[_SENT].set(bb_b)

  wl = jnp.concatenate([bb_W, jnp.zeros((_BB_DIM, _DIM), jnp.float32)], 0)
  wr = jnp.concatenate([jnp.zeros((_BB_DIM, _DIM), jnp.float32), bb_W], 0)
  pe = _pe_table(_L, _DIM)
  iota_k = jnp.broadcast_to(
      jnp.tile(jnp.arange(_DIM, dtype=jnp.float32), _RPB)[None, :],
      (_L, _RPB * _DIM))
  rones = jnp.repeat(jnp.eye(_RPB, dtype=jnp.float32), _DIM, axis=1)

  h, m = pl.pallas_call(
      _tc_body,
      grid=(_NB,),
      in_specs=[
          pl.BlockSpec((_RPB, _L), lambda i: (i, 0)),
          pl.BlockSpec((8 * _L, _DIM), lambda i: (i, 0)),
          pl.BlockSpec((_DIM, _DIM), lambda i: (0, 0)),
          pl.BlockSpec((_DIM, _DIM), lambda i: (0, 0)),
          pl.BlockSpec((_DIM, _DIM), lambda i: (0, 0)),
          pl.BlockSpec((_L, _DIM), lambda i: (0, 0)),
          pl.BlockSpec((_L, _RPB * _DIM), lambda i: (0, 0)),
          pl.BlockSpec((_RPB, _RPB * _DIM), lambda i: (0, 0)),
      ],
      out_specs=[
          pl.BlockSpec((_RPB, _L, _DIM), lambda i: (i, 0, 0)),
          pl.BlockSpec((_RPB, _L), lambda i: (i, 0)),
      ],
      out_shape=[
          jax.ShapeDtypeStruct((_B, _L, _DIM), jnp.float32),
          jax.ShapeDtypeStruct((_B, _L), jnp.float32),
      ],
  )(cidx, staging, comb, wl, wr, pe, iota_k, rones)
  return h, m


# speedup vs baseline: 13.0866x; 1.0005x over previous
import functools

import jax
import jax.numpy as jnp
from jax import lax
from jax.experimental import pallas as pl
from jax.experimental.pallas import tpu as pltpu
from jax.experimental.pallas import tpu_sc as plsc

_DIM = 128
_BB_DIM = 64
_B, _L = 4096, 200
_N = _B * _L
_NC, _NS = 2, 16
_NW = _NC * _NS
_CHUNK = 128
_NSTG = _N // 2
_PER_W_STG = _NSTG // _NW
_NCHS = _PER_W_STG // _CHUNK
_NBUF_SC = 4
_RPB = 16
_NB = _B // _RPB
_SENT = 120


def _sc_gather(idx_left, idx_right, table):
  mesh = plsc.VectorSubcoreMesh(
      core_axis_name="c", subcore_axis_name="s",
      num_cores=_NC, num_subcores=_NS)

  @functools.partial(
      pl.kernel,
      out_type=jax.ShapeDtypeStruct((_NSTG, _DIM), jnp.float32),
      mesh=mesh,
      compiler_params=pltpu.CompilerParams(use_tc_tiling_on_sc=False),
      scratch_types=(
          [pltpu.VMEM((_NCHS, _CHUNK), jnp.int32),
           pltpu.VMEM((_NCHS, _CHUNK), jnp.int32),
           pltpu.VMEM((_NBUF_SC, _CHUNK, _BB_DIM), jnp.float32),
           pltpu.VMEM((_NBUF_SC, _CHUNK, _BB_DIM), jnp.float32)]
          + [pltpu.SemaphoreType.DMA] * _NBUF_SC
      ),
  )
  def gather_kernel(idxl_hbm, idxr_hbm, table_hbm, out_hbm,
                    idxl_v, idxr_v, bufl, bufr, *gsems):
    wid = lax.axis_index("s") * _NC + lax.axis_index("c")
    pltpu.sync_copy(idxl_hbm.at[pl.ds(wid * _NCHS, _NCHS)], idxl_v)
    pltpu.sync_copy(idxr_hbm.at[pl.ds(wid * _NCHS, _NCHS)], idxr_v)

    def start_gather(c, b):
      pltpu.make_async_copy(
          table_hbm.at[idxl_v.at[c]], bufl.at[b], gsems[b]).start()
      pltpu.make_async_copy(
          table_hbm.at[idxr_v.at[c]], bufr.at[b], gsems[b]).start()

    def wait_gather(c, b):
      pltpu.make_async_copy(
          table_hbm.at[idxl_v.at[c]], bufl.at[b], gsems[b]).wait()
      pltpu.make_async_copy(
          table_hbm.at[idxr_v.at[c]], bufr.at[b], gsems[b]).wait()

    stg0 = wid * _PER_W_STG
    for b in range(_NBUF_SC):
      start_gather(b, b)

    def body(it, carry):
      c0 = it * _NBUF_SC
      for b in range(_NBUF_SC):
        c = c0 + b
        wait_gather(c, b)
        j0 = stg0 + c * _CHUNK
        pltpu.sync_copy(
            bufl.at[b], out_hbm.at[pl.ds(j0, _CHUNK), pl.ds(0, _BB_DIM)])
        pltpu.sync_copy(
            bufr.at[b],
            out_hbm.at[pl.ds(j0, _CHUNK), pl.ds(_BB_DIM, _BB_DIM)])

        @pl.when(c + _NBUF_SC < _NCHS)
        def _():
          start_gather(c + _NBUF_SC, b)
      return carry

    lax.fori_loop(0, _NCHS // _NBUF_SC, body, 0)

  return gather_kernel(idx_left, idx_right, table)


def _tc_body(cid_ref, stg_ref, comb_ref, wl_ref, wr_ref, pe_ref,
             iota_ref, rones_ref, h_ref, m_ref):
  cid_nat = cid_ref[...]
  m_ref[...] = jnp.where(cid_nat != 0, 0.0, -jnp.inf).astype(jnp.float32)

  cidT = jnp.transpose(cid_nat.astype(jnp.float32))
  bcast = jnp.dot(cidT, rones_ref[...],
                  preferred_element_type=jnp.float32)
  iota = iota_ref[...]

  stg = stg_ref[...]
  projl = jnp.dot(stg, wl_ref[...], preferred_element_type=jnp.float32)
  projr = jnp.dot(stg, wr_ref[...], preferred_element_type=jnp.float32)

  pe = pe_ref[...]
  comb = comb_ref[...]
  sent = jnp.float32(_SENT)
  for l in range(_RPB):
    onehot = (bcast[:, l * _DIM:(l + 1) * _DIM] == iota).astype(jnp.float32)
    base = jnp.dot(onehot, comb,
                   preferred_element_type=jnp.float32)
    proj = (projl[(l % 8) * _L:(l % 8 + 1) * _L, :] if l < 8
            else projr[(l - 8) * _L:(l - 7) * _L, :])
    is_bb = bcast[:, l * _DIM:(l + 1) * _DIM] == sent
    h_ref[l] = base + pe + jnp.where(is_bb, proj, 0.0)


def _pe_table(L, d):
  pos = jnp.arange(L, dtype=jnp.float32)[:, None]
  i = jnp.arange(0, d, 2, dtype=jnp.float32)[None, :]
  angle = pos / jnp.power(10000.0, i / d)
  pe = jnp.zeros((L, d), dtype=jnp.float32)
  pe = pe.at[:, 0::2].set(jnp.sin(angle))
  pe = pe.at[:, 1::2].set(jnp.cos(angle))
  return pe


def kernel(token_types, bb_indices, rxn_indices, token_table, bb_table,
           bb_W, bb_b, rxn_table):
  tt32 = token_types.astype(jnp.int32)
  rx32 = rxn_indices.astype(jnp.int32)
  cidx = jnp.where(tt32 == 2, rx32 + 8,
                   jnp.where(tt32 == 1, jnp.int32(_SENT), tt32))

  mi = bb_indices.astype(jnp.int32).reshape(_NB, 2, 8 * _L)
  idx_left = mi[:, 0].reshape(_NW * _NCHS, _CHUNK)
  idx_right = mi[:, 1].reshape(_NW * _NCHS, _CHUNK)

  staging = _sc_gather(idx_left, idx_right, bb_table)

  comb = jnp.zeros((_DIM, _DIM), jnp.float32)
  comb = comb.at[:8].set(token_table)
  comb = comb.at[8:109].set(rxn_table)
  comb = comb.at[_SENT].set(bb_b)

  wl = jnp.concatenate([bb_W, jnp.zeros((_BB_DIM, _DIM), jnp.float32)], 0)
  wr = jnp.concatenate([jnp.zeros((_BB_DIM, _DIM), jnp.float32), bb_W], 0)
  pe = _pe_table(_L, _DIM)
  iota_k = jnp.broadcast_to(
      jnp.arange(_DIM, dtype=jnp.float32)[None, :], (_L, _DIM))
  rones = jnp.repeat(jnp.eye(_RPB, dtype=jnp.float32), _DIM, axis=1)

  h, m = pl.pallas_call(
      _tc_body,
      grid=(_NB,),
      in_specs=[
          pl.BlockSpec((_RPB, _L), lambda i: (i, 0)),
          pl.BlockSpec((8 * _L, _DIM), lambda i: (i, 0)),
          pl.BlockSpec((_DIM, _DIM), lambda i: (0, 0)),
          pl.BlockSpec((_DIM, _DIM), lambda i: (0, 0)),
          pl.BlockSpec((_DIM, _DIM), lambda i: (0, 0)),
          pl.BlockSpec((_L, _DIM), lambda i: (0, 0)),
          pl.BlockSpec((_L, _DIM), lambda i: (0, 0)),
          pl.BlockSpec((_RPB, _RPB * _DIM), lambda i: (0, 0)),
      ],
      out_specs=[
          pl.BlockSpec((_RPB, _L, _DIM), lambda i: (i, 0, 0)),
          pl.BlockSpec((_RPB, _L), lambda i: (i, 0)),
      ],
      out_shape=[
          jax.ShapeDtypeStruct((_B, _L, _DIM), jnp.float32),
          jax.ShapeDtypeStruct((_B, _L), jnp.float32),
      ],
  )(cidx, staging, comb, wl, wr, pe, iota_k, rones)
  return h, m


# speedup vs baseline: 14.0238x; 1.0716x over previous
import functools

import jax
import jax.numpy as jnp
from jax import lax
from jax.experimental import pallas as pl
from jax.experimental.pallas import tpu as pltpu
from jax.experimental.pallas import tpu_sc as plsc

_DIM = 128
_BB_DIM = 64
_B, _L = 4096, 200
_N = _B * _L
_NC, _NS = 2, 16
_NW = _NC * _NS
_CHUNK = 128
_NSTG = _N // 2
_PER_W_STG = _NSTG // _NW
_NCHS = _PER_W_STG // _CHUNK
_NBUF_SC = 4
_RPB = 32
_NB = _B // _RPB
_SENT = 120


def _sc_gather(idx_left, idx_right, table):
  mesh = plsc.VectorSubcoreMesh(
      core_axis_name="c", subcore_axis_name="s",
      num_cores=_NC, num_subcores=_NS)

  @functools.partial(
      pl.kernel,
      out_type=jax.ShapeDtypeStruct((_NSTG, _DIM), jnp.float32),
      mesh=mesh,
      compiler_params=pltpu.CompilerParams(use_tc_tiling_on_sc=False),
      scratch_types=(
          [pltpu.VMEM((_NCHS, _CHUNK), jnp.int32),
           pltpu.VMEM((_NCHS, _CHUNK), jnp.int32),
           pltpu.VMEM((_NBUF_SC, _CHUNK, _BB_DIM), jnp.float32),
           pltpu.VMEM((_NBUF_SC, _CHUNK, _BB_DIM), jnp.float32)]
          + [pltpu.SemaphoreType.DMA] * _NBUF_SC
      ),
  )
  def gather_kernel(idxl_hbm, idxr_hbm, table_hbm, out_hbm,
                    idxl_v, idxr_v, bufl, bufr, *gsems):
    wid = lax.axis_index("s") * _NC + lax.axis_index("c")
    pltpu.sync_copy(idxl_hbm.at[pl.ds(wid * _NCHS, _NCHS)], idxl_v)
    pltpu.sync_copy(idxr_hbm.at[pl.ds(wid * _NCHS, _NCHS)], idxr_v)

    def start_gather(c, b):
      pltpu.make_async_copy(
          table_hbm.at[idxl_v.at[c]], bufl.at[b], gsems[b]).start()
      pltpu.make_async_copy(
          table_hbm.at[idxr_v.at[c]], bufr.at[b], gsems[b]).start()

    def wait_gather(c, b):
      pltpu.make_async_copy(
          table_hbm.at[idxl_v.at[c]], bufl.at[b], gsems[b]).wait()
      pltpu.make_async_copy(
          table_hbm.at[idxr_v.at[c]], bufr.at[b], gsems[b]).wait()

    stg0 = wid * _PER_W_STG
    for b in range(_NBUF_SC):
      start_gather(b, b)

    def body(it, carry):
      c0 = it * _NBUF_SC
      for b in range(_NBUF_SC):
        c = c0 + b
        wait_gather(c, b)
        j0 = stg0 + c * _CHUNK
        pltpu.sync_copy(
            bufl.at[b], out_hbm.at[pl.ds(j0, _CHUNK), pl.ds(0, _BB_DIM)])
        pltpu.sync_copy(
            bufr.at[b],
            out_hbm.at[pl.ds(j0, _CHUNK), pl.ds(_BB_DIM, _BB_DIM)])

        @pl.when(c + _NBUF_SC < _NCHS)
        def _():
          start_gather(c + _NBUF_SC, b)
      return carry

    lax.fori_loop(0, _NCHS // _NBUF_SC, body, 0)

  return gather_kernel(idx_left, idx_right, table)


def _tc_body(cid_ref, stg_ref, comb_ref, wl_ref, wr_ref, pe_ref,
             iota_ref, rones_ref, h_ref, m_ref):
  cid_nat = cid_ref[...]
  m_ref[...] = jnp.where(cid_nat != 0, 0.0, -jnp.inf).astype(jnp.float32)

  cidT = jnp.transpose(cid_nat.astype(jnp.float32))
  bcast = jnp.dot(cidT, rones_ref[...],
                  preferred_element_type=jnp.float32)
  iota = iota_ref[...]

  stg = stg_ref[...]
  projl = jnp.dot(stg, wl_ref[...], preferred_element_type=jnp.float32)
  projr = jnp.dot(stg, wr_ref[...], preferred_element_type=jnp.float32)
  hb = _RPB // 2

  pe = pe_ref[...]
  comb = comb_ref[...]
  sent = jnp.float32(_SENT)
  for l in range(_RPB):
    onehot = (bcast[:, l * _DIM:(l + 1) * _DIM] == iota).astype(jnp.float32)
    base = jnp.dot(onehot, comb,
                   preferred_element_type=jnp.float32)
    proj = (projl[l * _L:(l + 1) * _L, :] if l < hb
            else projr[(l - hb) * _L:(l - hb + 1) * _L, :])
    is_bb = bcast[:, l * _DIM:(l + 1) * _DIM] == sent
    h_ref[l] = base + pe + jnp.where(is_bb, proj, 0.0)


def _pe_table(L, d):
  pos = jnp.arange(L, dtype=jnp.float32)[:, None]
  i = jnp.arange(0, d, 2, dtype=jnp.float32)[None, :]
  angle = pos / jnp.power(10000.0, i / d)
  pe = jnp.zeros((L, d), dtype=jnp.float32)
  pe = pe.at[:, 0::2].set(jnp.sin(angle))
  pe = pe.at[:, 1::2].set(jnp.cos(angle))
  return pe


def kernel(token_types, bb_indices, rxn_indices, token_table, bb_table,
           bb_W, bb_b, rxn_table):
  tt32 = token_types.astype(jnp.int32)
  rx32 = rxn_indices.astype(jnp.int32)
  cidx = jnp.where(tt32 == 2, rx32 + 8,
                   jnp.where(tt32 == 1, jnp.int32(_SENT), tt32))

  mi = bb_indices.astype(jnp.int32).reshape(_NB, 2, (_RPB // 2) * _L)
  idx_left = mi[:, 0].reshape(_NW * _NCHS, _CHUNK)
  idx_right = mi[:, 1].reshape(_NW * _NCHS, _CHUNK)

  staging = _sc_gather(idx_left, idx_right, bb_table)

  comb = jnp.zeros((_DIM, _DIM), jnp.float32)
  comb = comb.at[:8].set(token_table)
  comb = comb.at[8:109].set(rxn_table)
  comb = comb.at[_SENT].set(bb_b)

  wl = jnp.concatenate([bb_W, jnp.zeros((_BB_DIM, _DIM), jnp.float32)], 0)
  wr = jnp.concatenate([jnp.zeros((_BB_DIM, _DIM), jnp.float32), bb_W], 0)
  pe = _pe_table(_L, _DIM)
  iota_k = jnp.broadcast_to(
      jnp.arange(_DIM, dtype=jnp.float32)[None, :], (_L, _DIM))
  rones = jnp.repeat(jnp.eye(_RPB, dtype=jnp.float32), _DIM, axis=1)

  h, m = pl.pallas_call(
      _tc_body,
      grid=(_NB,),
      in_specs=[
          pl.BlockSpec((_RPB, _L), lambda i: (i, 0)),
          pl.BlockSpec(((_RPB // 2) * _L, _DIM), lambda i: (i, 0)),
          pl.BlockSpec((_DIM, _DIM), lambda i: (0, 0)),
          pl.BlockSpec((_DIM, _DIM), lambda i: (0, 0)),
          pl.BlockSpec((_DIM, _DIM), lambda i: (0, 0)),
          pl.BlockSpec((_L, _DIM), lambda i: (0, 0)),
          pl.BlockSpec((_L, _DIM), lambda i: (0, 0)),
          pl.BlockSpec((_RPB, _RPB * _DIM), lambda i: (0, 0)),
      ],
      out_specs=[
          pl.BlockSpec((_RPB, _L, _DIM), lambda i: (i, 0, 0)),
          pl.BlockSpec((_RPB, _L), lambda i: (i, 0)),
      ],
      out_shape=[
          jax.ShapeDtypeStruct((_B, _L, _DIM), jnp.float32),
          jax.ShapeDtypeStruct((_B, _L), jnp.float32),
      ],
  )(cidx, staging, comb, wl, wr, pe, iota_k, rones)
  return h, m
